# Initial kernel scaffold; baseline (speedup 1.0000x reference)
import jax
import jax.numpy as jnp
from jax.experimental import pallas as pl


def kernel(coordinates, features, W1, a1_src, a1_dst, b1, W2, a2_src, a2_dst, b2, W3, a3_src, a3_dst, b3):
    raise NotImplementedError("write your pallas kernel here")


# speedup vs baseline: 1.4654x; 1.4654x over previous
import functools

import jax
import jax.numpy as jnp
from jax.experimental import pallas as pl
from jax.experimental.pallas import tpu as pltpu

NEG_SLOPE = 0.2
K = 16


def _matmul_kernel(x_ref, w_ref, asrc_ref, adst_ref, h_ref, as_ref, ad_ref):
    h = jnp.dot(x_ref[...], w_ref[...], preferred_element_type=jnp.float32)
    h_ref[...] = h
    as_ref[...] = jnp.sum(h * asrc_ref[...], axis=-1, keepdims=True)
    ad_ref[...] = jnp.sum(h * adst_ref[...], axis=-1, keepdims=True)


def _project(x, W, att_src, att_dst):
    N, IN = x.shape
    OUT = W.shape[1]
    BN = 1000
    grid = (N // BN,)
    h, a_s, a_d = pl.pallas_call(
        _matmul_kernel,
        grid=grid,
        in_specs=[
            pl.BlockSpec((BN, IN), lambda i: (i, 0)),
            pl.BlockSpec((IN, OUT), lambda i: (0, 0)),
            pl.BlockSpec((1, OUT), lambda i: (0, 0)),
            pl.BlockSpec((1, OUT), lambda i: (0, 0)),
        ],
        out_specs=[
            pl.BlockSpec((BN, OUT), lambda i: (i, 0)),
            pl.BlockSpec((BN, 1), lambda i: (i, 0)),
            pl.BlockSpec((BN, 1), lambda i: (i, 0)),
        ],
        out_shape=[
            jax.ShapeDtypeStruct((N, OUT), jnp.float32),
            jax.ShapeDtypeStruct((N, 1), jnp.float32),
            jax.ShapeDtypeStruct((N, 1), jnp.float32),
        ],
    )(x, W, att_src.reshape(1, OUT), att_dst.reshape(1, OUT))
    return h, a_s[:, 0], a_d[:, 0]


def _knn(x):
    inner = -2.0 * jnp.matmul(jnp.transpose(x, (0, 2, 1)), x)
    xx = jnp.sum(x ** 2, axis=1, keepdims=True)
    pairwise_distance = -xx - inner - jnp.transpose(xx, (0, 2, 1))
    return jax.lax.top_k(pairwise_distance, K)[1]


def _gat_layer(x, idx, W, att_src, att_dst, bias):
    N = x.shape[0]
    h, a_src, a_dst = _project(x, W, att_src, att_dst)
    M = jnp.maximum(jnp.max(a_src) + jnp.max(a_dst), 0.0)
    alpha1 = a_src[idx] + a_dst[:, None]
    alpha1 = jnp.where(alpha1 > 0, alpha1, NEG_SLOPE * alpha1)
    p = jnp.exp(alpha1 - M)
    alpha2 = a_src[:, None] + a_dst[idx]
    alpha2 = jnp.where(alpha2 > 0, alpha2, NEG_SLOPE * alpha2)
    q = jnp.exp(alpha2 - M)

    denom = jnp.sum(p, axis=1)
    denom = denom + jax.ops.segment_sum(q.reshape(-1), idx.reshape(-1), num_segments=N)

    num = jnp.einsum("nk,nkc->nc", p, h[idx])
    num = num + jax.ops.segment_sum(
        (q[:, :, None] * h[:, None, :]).reshape(N * K, -1),
        idx.reshape(-1), num_segments=N)
    return num / (denom[:, None] + 1e-16) + bias


def kernel(coordinates, features, W1, a1_src, a1_dst, b1, W2, a2_src, a2_dst, b2, W3, a3_src, a3_dst, b3):
    B, _, N = coordinates.shape
    idx = _knn(coordinates)[0]
    x = jnp.transpose(features[0], (1, 0))
    g = _gat_layer(x, idx, W1, a1_src, a1_dst, b1)
    g = _gat_layer(g, idx, W2, a2_src, a2_dst, b2)
    g = _gat_layer(g, idx, W3, a3_src, a3_dst, b3)
    return jnp.transpose(g, (1, 0)).reshape(B, -1, N)


# speedup vs baseline: 6.0615x; 4.1363x over previous
import functools

import jax
import jax.numpy as jnp
from jax.experimental import pallas as pl
from jax.experimental.pallas import tpu as pltpu

NEG_SLOPE = 0.2
K = 16


def _matmul_kernel(x_ref, w_ref, asrc_ref, adst_ref, h_ref, as_ref, ad_ref):
    h = jnp.dot(x_ref[...], w_ref[...], preferred_element_type=jnp.float32)
    h_ref[...] = h
    as_ref[...] = jnp.sum(h * asrc_ref[...], axis=-1, keepdims=True)
    ad_ref[...] = jnp.sum(h * adst_ref[...], axis=-1, keepdims=True)


def _project(x, W, att_src, att_dst):
    N, IN = x.shape
    OUT = W.shape[1]
    BN = 1000
    grid = (N // BN,)
    h, a_s, a_d = pl.pallas_call(
        _matmul_kernel,
        grid=grid,
        in_specs=[
            pl.BlockSpec((BN, IN), lambda i: (i, 0)),
            pl.BlockSpec((IN, OUT), lambda i: (0, 0)),
            pl.BlockSpec((1, OUT), lambda i: (0, 0)),
            pl.BlockSpec((1, OUT), lambda i: (0, 0)),
        ],
        out_specs=[
            pl.BlockSpec((BN, OUT), lambda i: (i, 0)),
            pl.BlockSpec((BN, 1), lambda i: (i, 0)),
            pl.BlockSpec((BN, 1), lambda i: (i, 0)),
        ],
        out_shape=[
            jax.ShapeDtypeStruct((N, OUT), jnp.float32),
            jax.ShapeDtypeStruct((N, 1), jnp.float32),
            jax.ShapeDtypeStruct((N, 1), jnp.float32),
        ],
    )(x, W, att_src.reshape(1, OUT), att_dst.reshape(1, OUT))
    return h, a_s[:, 0], a_d[:, 0]


_KNN_NEG = -3.0e38


def _knn_kernel(xr_ref, x_ref, xxr_ref, xx_ref, idx_ref):
    inner = -2.0 * jnp.dot(xr_ref[...].T, x_ref[...], preferred_element_type=jnp.float32)
    dist = -xxr_ref[...] - inner - xx_ref[...]
    BR, NP = dist.shape
    iota = jax.lax.broadcasted_iota(jnp.int32, (BR, NP), 1)

    def body(t, dist):
        am = jnp.argmax(dist, axis=1).astype(jnp.int32)
        idx_ref[:, t] = am
        return jnp.where(iota == am[:, None], _KNN_NEG, dist)

    jax.lax.fori_loop(0, K, body, dist, unroll=True)


def _knn(x):
    _, N = x.shape
    NP = 10240
    BR = 256
    pad = jnp.full((3, NP - N), 1.0e4, jnp.float32)
    xp = jnp.concatenate([x, pad], axis=1)
    xx = jnp.sum(xp * xp, axis=0)
    idx = pl.pallas_call(
        _knn_kernel,
        grid=(NP // BR,),
        in_specs=[
            pl.BlockSpec((3, BR), lambda i: (0, i)),
            pl.BlockSpec((3, NP), lambda i: (0, 0)),
            pl.BlockSpec((BR, 1), lambda i: (i, 0)),
            pl.BlockSpec((1, NP), lambda i: (0, 0)),
        ],
        out_specs=pl.BlockSpec((BR, K), lambda i: (i, 0)),
        out_shape=jax.ShapeDtypeStruct((NP, K), jnp.int32),
    )(xp, xp, xx.reshape(NP, 1), xx.reshape(1, NP))
    return idx[:N]


def _gat_layer(x, idx, W, att_src, att_dst, bias):
    N = x.shape[0]
    h, a_src, a_dst = _project(x, W, att_src, att_dst)
    M = jnp.maximum(jnp.max(a_src) + jnp.max(a_dst), 0.0)
    alpha1 = a_src[idx] + a_dst[:, None]
    alpha1 = jnp.where(alpha1 > 0, alpha1, NEG_SLOPE * alpha1)
    p = jnp.exp(alpha1 - M)
    alpha2 = a_src[:, None] + a_dst[idx]
    alpha2 = jnp.where(alpha2 > 0, alpha2, NEG_SLOPE * alpha2)
    q = jnp.exp(alpha2 - M)

    denom = jnp.sum(p, axis=1)
    denom = denom + jax.ops.segment_sum(q.reshape(-1), idx.reshape(-1), num_segments=N)

    num = jnp.einsum("nk,nkc->nc", p, h[idx])
    num = num + jax.ops.segment_sum(
        (q[:, :, None] * h[:, None, :]).reshape(N * K, -1),
        idx.reshape(-1), num_segments=N)
    return num / (denom[:, None] + 1e-16) + bias


def kernel(coordinates, features, W1, a1_src, a1_dst, b1, W2, a2_src, a2_dst, b2, W3, a3_src, a3_dst, b3):
    B, _, N = coordinates.shape
    idx = _knn(coordinates[0])
    x = jnp.transpose(features[0], (1, 0))
    g = _gat_layer(x, idx, W1, a1_src, a1_dst, b1)
    g = _gat_layer(g, idx, W2, a2_src, a2_dst, b2)
    g = _gat_layer(g, idx, W3, a3_src, a3_dst, b3)
    return jnp.transpose(g, (1, 0)).reshape(B, -1, N)


# speedup vs baseline: 16.8542x; 2.7805x over previous
import functools

import jax
import jax.numpy as jnp
from jax import lax
from jax.experimental import pallas as pl
from jax.experimental.pallas import tpu as pltpu
from jax.experimental.pallas import tpu_sc as plsc

NEG_SLOPE = 0.2
K = 16
NR = 10240
HALF = NR // 2
NT = 16
TA = HALF // NT
_KNN_NEG = -3.0e38
_SC_PARAMS = pltpu.CompilerParams(needs_layout_passes=False, use_tc_tiling_on_sc=False)


def _knn_kernel(xr_ref, x_ref, xxr_ref, xx_ref, idx_ref):
    inner = -2.0 * jnp.dot(xr_ref[...].T, x_ref[...], preferred_element_type=jnp.float32)
    dist = -xxr_ref[...] - inner - xx_ref[...]
    BR, NP = dist.shape
    iota = lax.broadcasted_iota(jnp.int32, (BR, NP), 1)

    def body(t, dist):
        am = jnp.argmax(dist, axis=1).astype(jnp.int32)
        idx_ref[:, t] = am
        return jnp.where(iota == am[:, None], _KNN_NEG, dist)

    lax.fori_loop(0, K, body, dist, unroll=True)


def _knn(x):
    _, N = x.shape
    BR = 256
    pad = jnp.full((3, NR - N), 1.0e4, jnp.float32)
    xp = jnp.concatenate([x, pad], axis=1)
    xx = jnp.sum(xp * xp, axis=0)
    idx = pl.pallas_call(
        _knn_kernel,
        grid=(NR // BR,),
        in_specs=[
            pl.BlockSpec((3, BR), lambda i: (0, i)),
            pl.BlockSpec((3, NR), lambda i: (0, 0)),
            pl.BlockSpec((BR, 1), lambda i: (i, 0)),
            pl.BlockSpec((1, NR), lambda i: (0, 0)),
        ],
        out_specs=pl.BlockSpec((BR, K), lambda i: (i, 0)),
        out_shape=jax.ShapeDtypeStruct((NR, K), jnp.int32),
    )(xp, xp, xx.reshape(NR, 1), xx.reshape(1, NR))
    return idx


def _proj_kernel(x_ref, w_ref, asrc_ref, adst_ref, h_ref, as_ref, ad_ref, ms_ref, md_ref):
    i = pl.program_id(0)
    h = jnp.dot(x_ref[...], w_ref[...], preferred_element_type=jnp.float32)
    h_ref[...] = h
    a_s = jnp.sum(h * asrc_ref[...], axis=-1, keepdims=True)
    a_d = jnp.sum(h * adst_ref[...], axis=-1, keepdims=True)
    as_ref[...] = a_s
    ad_ref[...] = a_d

    @pl.when(i == 0)
    def _():
        ms_ref[...] = jnp.full((1, 1), -3.0e38, jnp.float32)
        md_ref[...] = jnp.full((1, 1), -3.0e38, jnp.float32)

    ms_ref[...] = jnp.maximum(ms_ref[...], jnp.max(a_s).reshape(1, 1))
    md_ref[...] = jnp.maximum(md_ref[...], jnp.max(a_d).reshape(1, 1))


def _project(x, W, att_src, att_dst):
    N, IN = x.shape
    OUT = W.shape[1]
    BN = 1024
    h, a_s, a_d, ms, md = pl.pallas_call(
        _proj_kernel,
        grid=(N // BN,),
        in_specs=[
            pl.BlockSpec((BN, IN), lambda i: (i, 0)),
            pl.BlockSpec((IN, OUT), lambda i: (0, 0)),
            pl.BlockSpec((1, OUT), lambda i: (0, 0)),
            pl.BlockSpec((1, OUT), lambda i: (0, 0)),
        ],
        out_specs=[
            pl.BlockSpec((BN, OUT), lambda i: (i, 0)),
            pl.BlockSpec((BN, 1), lambda i: (i, 0)),
            pl.BlockSpec((BN, 1), lambda i: (i, 0)),
            pl.BlockSpec((1, 1), lambda i: (0, 0)),
            pl.BlockSpec((1, 1), lambda i: (0, 0)),
        ],
        out_shape=[
            jax.ShapeDtypeStruct((N, OUT), jnp.float32),
            jax.ShapeDtypeStruct((N, 1), jnp.float32),
            jax.ShapeDtypeStruct((N, 1), jnp.float32),
            jax.ShapeDtypeStruct((1, 1), jnp.float32),
            jax.ShapeDtypeStruct((1, 1), jnp.float32),
        ],
    )(x, W, att_src.reshape(1, OUT), att_dst.reshape(1, OUT))
    return h, a_s[:, 0], a_d[:, 0], ms[0, 0], md[0, 0]


def _lrelu(z):
    return jnp.where(z > 0, z, NEG_SLOPE * z)


@jax.jit
def _sc_pq(idx, asrc, adst, mv):
    mesh = plsc.VectorSubcoreMesh(core_axis_name="c", subcore_axis_name="s")
    TB = NR // 32

    @functools.partial(
        pl.kernel,
        out_type=[
            jax.ShapeDtypeStruct((NR, K), jnp.float32),
            jax.ShapeDtypeStruct((NR, K), jnp.float32),
        ],
        mesh=mesh,
        compiler_params=_SC_PARAMS,
        scratch_types=[
            pltpu.VMEM((NR,), jnp.float32),
            pltpu.VMEM((NR,), jnp.float32),
            pltpu.VMEM((TB, 16), jnp.int32),
            pltpu.VMEM((16, 16), jnp.float32),
            pltpu.VMEM((16, 16), jnp.float32),
            pltpu.VMEM((16,), jnp.float32),
        ],
    )
    def pq_kernel(idx_hbm, asrc_hbm, adst_hbm, mv_hbm, p_hbm, q_hbm,
                  asrc_v, adst_v, idxv, ps, qs, mvv):
        c = lax.axis_index("c")
        s = lax.axis_index("s")
        base = pl.multiple_of((c * NT + s) * TB, 64)
        pltpu.sync_copy(asrc_hbm, asrc_v)
        pltpu.sync_copy(adst_hbm, adst_v)
        pltpu.sync_copy(mv_hbm, mvv)
        pltpu.sync_copy(idx_hbm.at[pl.ds(base, TB)], idxv)
        M = mvv[...]

        def body(t, _):
            n = base + t
            t16 = t % 16
            iv = idxv[t]
            nn = jnp.full((16,), n, jnp.int32)
            asg = plsc.load_gather(asrc_v, [iv])
            adg = plsc.load_gather(adst_v, [iv])
            asn = plsc.load_gather(asrc_v, [nn])
            adn = plsc.load_gather(adst_v, [nn])
            ps[t16, pl.ds(0, 16)] = jnp.exp(_lrelu(asg + adn) - M)
            qs[t16, pl.ds(0, 16)] = jnp.exp(_lrelu(asn + adg) - M)

            @pl.when(t16 == 15)
            def _():
                b = pl.multiple_of(base + t - 15, 16)
                pltpu.sync_copy(ps, p_hbm.at[pl.ds(b, 16)])
                pltpu.sync_copy(qs, q_hbm.at[pl.ds(b, 16)])

            return 0

        lax.fori_loop(0, TB, body, 0)

    return pq_kernel(idx, asrc, adst, mv)


@functools.partial(jax.jit, static_argnames=("D",))
def _sc_edge(idx, p, q, h, bias, D):
    DW = D + 16
    NV = D // 16
    mesh = plsc.VectorSubcoreMesh(core_axis_name="c", subcore_axis_name="s")

    @functools.partial(
        pl.kernel,
        out_type=jax.ShapeDtypeStruct((NR, D), jnp.float32),
        mesh=mesh,
        compiler_params=_SC_PARAMS,
        scratch_types=[
            pltpu.VMEM_SHARED((HALF + 1, DW), jnp.float32),
            pltpu.VMEM((16, 16), jnp.int32),
            pltpu.VMEM((16, 16), jnp.float32),
            pltpu.VMEM((16, 16), jnp.float32),
            pltpu.VMEM((64, D), jnp.float32),
            pltpu.VMEM((16, D), jnp.float32),
            pltpu.VMEM((16, DW), jnp.float32),
            pltpu.VMEM((16, DW), jnp.float32),
            pltpu.VMEM((D,), jnp.float32),
            pltpu.VMEM((16,), jnp.float32),
            pltpu.SemaphoreType.DMA,
            pltpu.SemaphoreType.DMA,
            pltpu.SemaphoreType.DMA,
        ],
    )
    def edge_kernel(idx_hbm, p_hbm, q_hbm, h_hbm, b_hbm, out_hbm,
                    acc, idxb, pb, qb, hbuf, gbuf, stage, e1buf, bv, rbuf,
                    gsem, ssem, esem):
        c = lax.axis_index("c")
        s = lax.axis_index("s")
        own0 = pl.multiple_of(c * HALF + s * TA, 64)
        oth0 = pl.multiple_of((1 - c) * HALF + s * TA, 64)
        ownl = pl.multiple_of(s * TA, 64)
        iota16 = lax.broadcasted_iota(jnp.int32, (16,), 0)
        one0 = jnp.where(iota16 == 0, 1.0, 0.0)
        zeros16 = jnp.zeros((16,), jnp.float32)

        pltpu.sync_copy(b_hbm, bv)
        for r in range(16):
            for v in range(DW // 16):
                stage[r, pl.ds(v * 16, 16)] = zeros16
        for z in range(TA // 16):
            pltpu.sync_copy(stage, acc.at[pl.ds(ownl + z * 16, 16)])

        @pl.when(s == NT - 1)
        def _():
            pltpu.sync_copy(stage.at[pl.ds(0, 1)], acc.at[pl.ds(HALF, 1)])

        plsc.subcore_barrier()

        def body_a(t, _):
            t16 = t % 16

            @pl.when(t % 64 == 0)
            def _():
                pltpu.sync_copy(h_hbm.at[pl.ds(pl.multiple_of(own0 + t, 64), 64)], hbuf)

            @pl.when(t16 == 0)
            def _():
                b = pl.multiple_of(own0 + t, 16)
                pltpu.sync_copy(idx_hbm.at[pl.ds(b, 16)], idxb)
                pltpu.sync_copy(p_hbm.at[pl.ds(b, 16)], pb)
                pltpu.sync_copy(q_hbm.at[pl.ds(b, 16)], qb)

            iv = idxb[t16]
            tt = jnp.full((16,), t16, jnp.int32)
            pltpu.async_copy(h_hbm.at[iv], gbuf, gsem).wait()
            accv = [zeros16 for _ in range(NV)]
            for k in range(16):
                pk = plsc.load_gather(pb, [tt, jnp.full((16,), k, jnp.int32)])
                for v in range(NV):
                    accv[v] = accv[v] + pk * gbuf[k, pl.ds(v * 16, 16)]
            for v in range(NV):
                e1buf[t16, pl.ds(v * 16, 16)] = accv[v]
            e1buf[t16, pl.ds(D, 16)] = jnp.sum(pb[t16]) * one0
            hv = [hbuf[t % 64, pl.ds(v * 16, 16)] for v in range(NV)]
            for k in range(16):
                qk = plsc.load_gather(qb, [tt, jnp.full((16,), k, jnp.int32)])
                for v in range(NV):
                    stage[k, pl.ds(v * 16, 16)] = qk * hv[v]
                stage[k, pl.ds(D, 16)] = qk * one0
            local = iv - c * HALF
            valid = (local >= 0) & (local < HALF)
            sidx = jnp.where(valid, local, HALF)
            pltpu.async_copy(stage, acc.at[sidx], ssem, add=True).wait()

            @pl.when(t16 == 15)
            def _():
                idv = (ownl + t - 15) + iota16
                pltpu.async_copy(e1buf, acc.at[idv], esem, add=True).wait()

            return 0

        lax.fori_loop(0, TA, body_a, 0)

        def body_b(t, _):
            t16 = t % 16

            @pl.when(t % 64 == 0)
            def _():
                pltpu.sync_copy(h_hbm.at[pl.ds(pl.multiple_of(oth0 + t, 64), 64)], hbuf)

            @pl.when(t16 == 0)
            def _():
                b = pl.multiple_of(oth0 + t, 16)
                pltpu.sync_copy(idx_hbm.at[pl.ds(b, 16)], idxb)
                pltpu.sync_copy(q_hbm.at[pl.ds(b, 16)], qb)

            iv = idxb[t16]
            tt = jnp.full((16,), t16, jnp.int32)
            hv = [hbuf[t % 64, pl.ds(v * 16, 16)] for v in range(NV)]
            for k in range(16):
                qk = plsc.load_gather(qb, [tt, jnp.full((16,), k, jnp.int32)])
                for v in range(NV):
                    stage[k, pl.ds(v * 16, 16)] = qk * hv[v]
                stage[k, pl.ds(D, 16)] = qk * one0
            local = iv - c * HALF
            valid = (local >= 0) & (local < HALF)
            sidx = jnp.where(valid, local, HALF)
            pltpu.async_copy(stage, acc.at[sidx], ssem, add=True).wait()
            return 0

        lax.fori_loop(0, TA, body_b, 0)
        plsc.subcore_barrier()

        bias = [bv[pl.ds(v * 16, 16)] for v in range(NV)]

        def body_f(f, _):
            basel = pl.multiple_of(ownl + f * 16, 16)
            baseg = pl.multiple_of(own0 + f * 16, 16)
            pltpu.sync_copy(acc.at[pl.ds(basel, 16)], stage)
            for r in range(16):
                dv = stage[r, pl.ds(D, 16)]
                rr = jnp.sum(one0 / (dv + 1e-16))
                for v in range(NV):
                    gbuf[r, pl.ds(v * 16, 16)] = stage[r, pl.ds(v * 16, 16)] * rr + bias[v]
            pltpu.sync_copy(gbuf, out_hbm.at[pl.ds(baseg, 16)])
            return 0

        lax.fori_loop(0, TA // 16, body_f, 0)

    return edge_kernel(idx, p, q, h, bias)


def _gat_layer(x, idx, W, att_src, att_dst, bias):
    h, a_src, a_dst, ms, md = _project(x, W, att_src, att_dst)
    Z = ms + md
    M = jnp.where(Z > 0, Z, NEG_SLOPE * Z)
    mv = jnp.full((16,), M, jnp.float32)
    p, q = _sc_pq(idx, a_src, a_dst, mv)
    return _sc_edge(idx, p, q, h, bias, D=W.shape[1])


def kernel(coordinates, features, W1, a1_src, a1_dst, b1, W2, a2_src, a2_dst, b2, W3, a3_src, a3_dst, b3):
    B, _, N = coordinates.shape
    idx = _knn(coordinates[0])
    x = jnp.transpose(features[0], (1, 0))
    x = jnp.pad(x, ((0, NR - N), (0, 0)))
    g = _gat_layer(x, idx, W1, a1_src, a1_dst, b1)
    g = _gat_layer(g, idx, W2, a2_src, a2_dst, b2)
    g = _gat_layer(g, idx, W3, a3_src, a3_dst, b3)
    return jnp.transpose(g[:N], (1, 0)).reshape(B, -1, N)


# speedup vs baseline: 17.0900x; 1.0140x over previous
import functools

import jax
import jax.numpy as jnp
from jax import lax
from jax.experimental import pallas as pl
from jax.experimental.pallas import tpu as pltpu
from jax.experimental.pallas import tpu_sc as plsc

NEG_SLOPE = 0.2
K = 16
NR = 10240
HALF = NR // 2
NT = 16
TA = HALF // NT
_KNN_NEG = -3.0e38
_SC_PARAMS = pltpu.CompilerParams(needs_layout_passes=False, use_tc_tiling_on_sc=False)


def _knn_kernel(xr_ref, x_ref, xxr_ref, xx_ref, idx_ref):
    inner = -2.0 * jnp.dot(xr_ref[...].T, x_ref[...], preferred_element_type=jnp.float32)
    dist = -xxr_ref[...] - inner - xx_ref[...]
    BR, NP = dist.shape
    iota = lax.broadcasted_iota(jnp.int32, (BR, NP), 1)

    def body(t, dist):
        am = jnp.argmax(dist, axis=1).astype(jnp.int32)
        idx_ref[:, t] = am
        return jnp.where(iota == am[:, None], _KNN_NEG, dist)

    lax.fori_loop(0, K, body, dist, unroll=True)


def _knn(x):
    _, N = x.shape
    BR = 256
    pad = jnp.full((3, NR - N), 1.0e4, jnp.float32)
    xp = jnp.concatenate([x, pad], axis=1)
    xx = jnp.sum(xp * xp, axis=0)
    idx = pl.pallas_call(
        _knn_kernel,
        grid=(NR // BR,),
        in_specs=[
            pl.BlockSpec((3, BR), lambda i: (0, i)),
            pl.BlockSpec((3, NR), lambda i: (0, 0)),
            pl.BlockSpec((BR, 1), lambda i: (i, 0)),
            pl.BlockSpec((1, NR), lambda i: (0, 0)),
        ],
        out_specs=pl.BlockSpec((BR, K), lambda i: (i, 0)),
        out_shape=jax.ShapeDtypeStruct((NR, K), jnp.int32),
    )(xp, xp, xx.reshape(NR, 1), xx.reshape(1, NR))
    return idx


def _proj_kernel(x_ref, w_ref, asrc_ref, adst_ref, h_ref, as_ref, ad_ref, ms_ref, md_ref):
    i = pl.program_id(0)
    h = jnp.dot(x_ref[...], w_ref[...], preferred_element_type=jnp.float32)
    h_ref[...] = h
    a_s = jnp.sum(h * asrc_ref[...], axis=-1, keepdims=True)
    a_d = jnp.sum(h * adst_ref[...], axis=-1, keepdims=True)
    as_ref[...] = a_s
    ad_ref[...] = a_d

    @pl.when(i == 0)
    def _():
        ms_ref[...] = jnp.full((1, 1), -3.0e38, jnp.float32)
        md_ref[...] = jnp.full((1, 1), -3.0e38, jnp.float32)

    ms_ref[...] = jnp.maximum(ms_ref[...], jnp.max(a_s).reshape(1, 1))
    md_ref[...] = jnp.maximum(md_ref[...], jnp.max(a_d).reshape(1, 1))


def _project(x, W, att_src, att_dst):
    N, IN = x.shape
    OUT = W.shape[1]
    BN = 1024
    h, a_s, a_d, ms, md = pl.pallas_call(
        _proj_kernel,
        grid=(N // BN,),
        in_specs=[
            pl.BlockSpec((BN, IN), lambda i: (i, 0)),
            pl.BlockSpec((IN, OUT), lambda i: (0, 0)),
            pl.BlockSpec((1, OUT), lambda i: (0, 0)),
            pl.BlockSpec((1, OUT), lambda i: (0, 0)),
        ],
        out_specs=[
            pl.BlockSpec((BN, OUT), lambda i: (i, 0)),
            pl.BlockSpec((BN, 1), lambda i: (i, 0)),
            pl.BlockSpec((BN, 1), lambda i: (i, 0)),
            pl.BlockSpec((1, 1), lambda i: (0, 0)),
            pl.BlockSpec((1, 1), lambda i: (0, 0)),
        ],
        out_shape=[
            jax.ShapeDtypeStruct((N, OUT), jnp.float32),
            jax.ShapeDtypeStruct((N, 1), jnp.float32),
            jax.ShapeDtypeStruct((N, 1), jnp.float32),
            jax.ShapeDtypeStruct((1, 1), jnp.float32),
            jax.ShapeDtypeStruct((1, 1), jnp.float32),
        ],
    )(x, W, att_src.reshape(1, OUT), att_dst.reshape(1, OUT))
    return h, a_s[:, 0], a_d[:, 0], ms[0, 0], md[0, 0]


def _lrelu(z):
    return jnp.where(z > 0, z, NEG_SLOPE * z)


@jax.jit
def _sc_pq(idx, asrc, adst, mv):
    mesh = plsc.VectorSubcoreMesh(core_axis_name="c", subcore_axis_name="s")
    TB = NR // 32

    @functools.partial(
        pl.kernel,
        out_type=[
            jax.ShapeDtypeStruct((NR, K), jnp.float32),
            jax.ShapeDtypeStruct((NR, K), jnp.float32),
        ],
        mesh=mesh,
        compiler_params=_SC_PARAMS,
        scratch_types=[
            pltpu.VMEM((NR,), jnp.float32),
            pltpu.VMEM((NR,), jnp.float32),
            pltpu.VMEM((TB, 16), jnp.int32),
            pltpu.VMEM((16, 16), jnp.float32),
            pltpu.VMEM((16, 16), jnp.float32),
            pltpu.VMEM((16,), jnp.float32),
        ],
    )
    def pq_kernel(idx_hbm, asrc_hbm, adst_hbm, mv_hbm, p_hbm, q_hbm,
                  asrc_v, adst_v, idxv, ps, qs, mvv):
        c = lax.axis_index("c")
        s = lax.axis_index("s")
        base = pl.multiple_of((c * NT + s) * TB, 64)
        pltpu.sync_copy(asrc_hbm, asrc_v)
        pltpu.sync_copy(adst_hbm, adst_v)
        pltpu.sync_copy(mv_hbm, mvv)
        pltpu.sync_copy(idx_hbm.at[pl.ds(base, TB)], idxv)
        M = mvv[...]

        def body(t, _):
            n = base + t
            t16 = t % 16
            iv = idxv[t]
            nn = jnp.full((16,), n, jnp.int32)
            asg = plsc.load_gather(asrc_v, [iv])
            adg = plsc.load_gather(adst_v, [iv])
            asn = plsc.load_gather(asrc_v, [nn])
            adn = plsc.load_gather(adst_v, [nn])
            ps[t16, pl.ds(0, 16)] = jnp.exp(_lrelu(asg + adn) - M)
            qs[t16, pl.ds(0, 16)] = jnp.exp(_lrelu(asn + adg) - M)

            @pl.when(t16 == 15)
            def _():
                b = pl.multiple_of(base + t - 15, 16)
                pltpu.sync_copy(ps, p_hbm.at[pl.ds(b, 16)])
                pltpu.sync_copy(qs, q_hbm.at[pl.ds(b, 16)])

            return 0

        lax.fori_loop(0, TB, body, 0)

    return pq_kernel(idx, asrc, adst, mv)


@functools.partial(jax.jit, static_argnames=("D",))
def _sc_edge(idx, p, q, h, bias, D):
    DW = D + 16
    NV = D // 16
    mesh = plsc.VectorSubcoreMesh(core_axis_name="c", subcore_axis_name="s")

    @functools.partial(
        pl.kernel,
        out_type=jax.ShapeDtypeStruct((NR, D), jnp.float32),
        mesh=mesh,
        compiler_params=_SC_PARAMS,
        scratch_types=[
            pltpu.VMEM_SHARED((HALF + 1, DW), jnp.float32),
            pltpu.VMEM((16, 16), jnp.int32),
            pltpu.VMEM((16, 16), jnp.float32),
            pltpu.VMEM((16, 16), jnp.float32),
            pltpu.VMEM((64, D), jnp.float32),
            pltpu.VMEM((32, D), jnp.float32),
            pltpu.VMEM((32, DW), jnp.float32),
            pltpu.VMEM((16, DW), jnp.float32),
            pltpu.VMEM((D,), jnp.float32),
            pltpu.SemaphoreType.DMA,
            pltpu.SemaphoreType.DMA,
            pltpu.SemaphoreType.DMA,
            pltpu.SemaphoreType.DMA,
            pltpu.SemaphoreType.DMA,
        ],
    )
    def edge_kernel(idx_hbm, p_hbm, q_hbm, h_hbm, b_hbm, out_hbm,
                    acc, idxb, pb, qb, hbuf, gbuf, stage, e1buf, bv,
                    gsem0, gsem1, ssem0, ssem1, esem):
        c = lax.axis_index("c")
        s = lax.axis_index("s")
        own0 = pl.multiple_of(c * HALF + s * TA, 64)
        oth0 = pl.multiple_of((1 - c) * HALF + s * TA, 64)
        ownl = pl.multiple_of(s * TA, 64)
        iota16 = lax.broadcasted_iota(jnp.int32, (16,), 0)
        one0 = jnp.where(iota16 == 0, 1.0, 0.0)
        zeros16 = jnp.zeros((16,), jnp.float32)
        gsems = (gsem0, gsem1)
        ssems = (ssem0, ssem1)

        pltpu.sync_copy(b_hbm, bv)
        for r in range(16):
            for v in range(DW // 16):
                stage[r, pl.ds(v * 16, 16)] = zeros16
        for z in range(TA // 16):
            pltpu.sync_copy(stage.at[pl.ds(0, 16)], acc.at[pl.ds(ownl + z * 16, 16)])

        @pl.when(s == NT - 1)
        def _():
            pltpu.sync_copy(stage.at[pl.ds(0, 1)], acc.at[pl.ds(HALF, 1)])

        plsc.subcore_barrier()

        def load_chunk(base, with_p):
            b = pl.multiple_of(base, 16)
            pltpu.sync_copy(idx_hbm.at[pl.ds(b, 16)], idxb)
            if with_p:
                pltpu.sync_copy(p_hbm.at[pl.ds(b, 16)], pb)
            pltpu.sync_copy(q_hbm.at[pl.ds(b, 16)], qb)

        def load_h(base):
            pltpu.sync_copy(h_hbm.at[pl.ds(pl.multiple_of(base, 64), 64)], hbuf)

        load_chunk(own0, True)
        load_h(own0)
        pltpu.async_copy(h_hbm.at[idxb[0]], gbuf.at[pl.ds(0, 16)], gsem0)

        def body_a(t2, _):
            for j in range(2):
                t = 2 * t2 + j
                slot = j
                g0 = slot * 16
                t16 = t % 16
                iv = idxb[t16]
                pltpu.make_async_copy(h_hbm.at[iv], gbuf.at[pl.ds(g0, 16)], gsems[j]).wait()
                tt = jnp.full((16,), t16, jnp.int32)
                accv = [zeros16 for _ in range(NV)]
                for k in range(16):
                    pk = plsc.load_gather(pb, [tt, jnp.full((16,), k, jnp.int32)])
                    for v in range(NV):
                        accv[v] = accv[v] + pk * gbuf[g0 + k, pl.ds(v * 16, 16)]
                @pl.when((t16 == 0) & (t >= 16))
                def _():
                    pltpu.make_async_copy(e1buf, acc.at[iv], esem).wait()

                for v in range(NV):
                    e1buf[t16, pl.ds(v * 16, 16)] = accv[v]
                e1buf[t16, pl.ds(D, 16)] = jnp.sum(pb[t16]) * one0
                @pl.when(t >= 2)
                def _():
                    pltpu.make_async_copy(stage.at[pl.ds(g0, 16)], acc.at[iv], ssems[j]).wait()

                hv = [hbuf[t % 64, pl.ds(v * 16, 16)] for v in range(NV)]
                for k in range(16):
                    qk = plsc.load_gather(qb, [tt, jnp.full((16,), k, jnp.int32)])
                    for v in range(NV):
                        stage[g0 + k, pl.ds(v * 16, 16)] = qk * hv[v]
                    stage[g0 + k, pl.ds(D, 16)] = qk * one0
                local = iv - c * HALF
                valid = (local >= 0) & (local < HALF)
                sidx = jnp.where(valid, local, HALF)
                pltpu.async_copy(stage.at[pl.ds(g0, 16)], acc.at[sidx], ssems[j], add=True)

                @pl.when(t16 == 15)
                def _():
                    idv = (ownl + t - 15) + iota16
                    pltpu.async_copy(e1buf, acc.at[idv], esem, add=True)

                @pl.when((t16 == 15) & (t < TA - 1))
                def _():
                    load_chunk(own0 + t + 1, True)

                @pl.when(((t + 1) % 64 == 0) & (t < TA - 1))
                def _():
                    load_h(own0 + t + 1)

                @pl.when(t < TA - 1)
                def _():
                    niv = idxb[(t + 1) % 16]
                    pltpu.async_copy(h_hbm.at[niv], gbuf.at[pl.ds((1 - slot) * 16, 16)],
                                     gsems[1 - j])

            return 0

        lax.fori_loop(0, TA // 2, body_a, 0)
        dummy = jnp.zeros((16,), jnp.int32)
        pltpu.make_async_copy(stage.at[pl.ds(0, 16)], acc.at[dummy], ssem0).wait()
        pltpu.make_async_copy(stage.at[pl.ds(16, 16)], acc.at[dummy], ssem1).wait()
        pltpu.make_async_copy(e1buf, acc.at[dummy], esem).wait()

        load_chunk(oth0, False)
        load_h(oth0)

        def body_b(t2, _):
            for j in range(2):
                t = 2 * t2 + j
                slot = j
                g0 = slot * 16
                t16 = t % 16
                iv = idxb[t16]
                tt = jnp.full((16,), t16, jnp.int32)

                @pl.when(t >= 2)
                def _():
                    pltpu.make_async_copy(stage.at[pl.ds(g0, 16)], acc.at[iv], ssems[j]).wait()

                hv = [hbuf[t % 64, pl.ds(v * 16, 16)] for v in range(NV)]
                for k in range(16):
                    qk = plsc.load_gather(qb, [tt, jnp.full((16,), k, jnp.int32)])
                    for v in range(NV):
                        stage[g0 + k, pl.ds(v * 16, 16)] = qk * hv[v]
                    stage[g0 + k, pl.ds(D, 16)] = qk * one0
                local = iv - c * HALF
                valid = (local >= 0) & (local < HALF)
                sidx = jnp.where(valid, local, HALF)
                pltpu.async_copy(stage.at[pl.ds(g0, 16)], acc.at[sidx], ssems[j], add=True)

                @pl.when((t16 == 15) & (t < TA - 1))
                def _():
                    load_chunk(oth0 + t + 1, False)

                @pl.when(((t + 1) % 64 == 0) & (t < TA - 1))
                def _():
                    load_h(oth0 + t + 1)

            return 0

        lax.fori_loop(0, TA // 2, body_b, 0)
        pltpu.make_async_copy(stage.at[pl.ds(0, 16)], acc.at[dummy], ssem0).wait()
        pltpu.make_async_copy(stage.at[pl.ds(16, 16)], acc.at[dummy], ssem1).wait()
        plsc.subcore_barrier()

        bias = [bv[pl.ds(v * 16, 16)] for v in range(NV)]

        def body_f(f, _):
            basel = pl.multiple_of(ownl + f * 16, 16)
            baseg = pl.multiple_of(own0 + f * 16, 16)
            pltpu.sync_copy(acc.at[pl.ds(basel, 16)], stage.at[pl.ds(0, 16)])
            for r in range(16):
                dv = stage[r, pl.ds(D, 16)]
                rr = jnp.sum(one0 / (dv + 1e-16))
                for v in range(NV):
                    gbuf[r, pl.ds(v * 16, 16)] = stage[r, pl.ds(v * 16, 16)] * rr + bias[v]
            pltpu.sync_copy(gbuf.at[pl.ds(0, 16)], out_hbm.at[pl.ds(baseg, 16)])
            return 0

        lax.fori_loop(0, TA // 16, body_f, 0)

    return edge_kernel(idx, p, q, h, bias)


def _gat_layer(x, idx, W, att_src, att_dst, bias):
    h, a_src, a_dst, ms, md = _project(x, W, att_src, att_dst)
    Z = ms + md
    M = jnp.where(Z > 0, Z, NEG_SLOPE * Z)
    mv = jnp.full((16,), M, jnp.float32)
    p, q = _sc_pq(idx, a_src, a_dst, mv)
    return _sc_edge(idx, p, q, h, bias, D=W.shape[1])


def kernel(coordinates, features, W1, a1_src, a1_dst, b1, W2, a2_src, a2_dst, b2, W3, a3_src, a3_dst, b3):
    B, _, N = coordinates.shape
    idx = _knn(coordinates[0])
    x = jnp.transpose(features[0], (1, 0))
    x = jnp.pad(x, ((0, NR - N), (0, 0)))
    g = _gat_layer(x, idx, W1, a1_src, a1_dst, b1)
    g = _gat_layer(g, idx, W2, a2_src, a2_dst, b2)
    g = _gat_layer(g, idx, W3, a3_src, a3_dst, b3)
    return jnp.transpose(g[:N], (1, 0)).reshape(B, -1, N)
